# Initial kernel scaffold; baseline (speedup 1.0000x reference)
import jax
import jax.numpy as jnp
from jax.experimental import pallas as pl


def kernel(atomic_numbers):
    raise NotImplementedError("write your pallas kernel here")


# speedup vs baseline: 3.9195x; 3.9195x over previous
import dataclasses
import functools

import numpy as np
import jax
import jax.numpy as jnp
from jax import lax
from jax.experimental import pallas as pl
from jax.experimental.pallas import tpu as pltpu
from jax.experimental.pallas import tpu_sc as plsc

N_ATOMS = 100000
MAX_Z = 20
N_COLS = 6
LANES = 16
NUM_WORKERS = 32

CHUNK = 3120
NGROUPS = CHUNK // LANES
REM_BASE = NUM_WORKERS * CHUNK
REM_ATOMS = N_ATOMS - REM_BASE
REM_NGROUPS = REM_ATOMS // LANES

_lut = np.full((MAX_Z,), 5, dtype=np.int32)
_lut[[6, 7, 8, 15, 16]] = [0, 1, 2, 3, 4]
_U = np.zeros((128,), dtype=np.float32)
for _z in range(MAX_Z):
    _U[N_COLS * _z + _lut[_z]] = 1.0

def _tile_body(a_ref, u_ref, o_ref, ngroups):
    iota = lax.iota(jnp.int32, LANES)
    qv = [(iota + LANES * v) // N_COLS for v in range(N_COLS)]
    rv = [(iota + LANES * v) - N_COLS * qv[v] for v in range(N_COLS)]

    @pl.loop(0, ngroups)
    def _(g):
        abase = g * LANES
        obase = g * (LANES * N_COLS)
        for v in range(N_COLS):
            a_q = plsc.load_gather(a_ref, [abase + qv[v]])
            pos = a_q * N_COLS + rv[v]
            vals = plsc.load_gather(u_ref, [pos])
            o_ref[pl.ds(obase + v * LANES, LANES)] = vals


def _sc_kernel(a_hbm, u_hbm, out_hbm, u_v, a_v, o_v):
    wid = lax.axis_index("s") * 2 + lax.axis_index("c")
    pltpu.sync_copy(u_hbm, u_v)
    base = wid * CHUNK
    pltpu.sync_copy(a_hbm.at[pl.ds(base, CHUNK)], a_v)
    _tile_body(a_v, u_v, o_v, NGROUPS)
    pltpu.sync_copy(o_v, out_hbm.at[pl.ds(base * N_COLS, CHUNK * N_COLS)])

    @pl.when(wid == 0)
    def _():
        pltpu.sync_copy(
            a_hbm.at[pl.ds(REM_BASE, REM_ATOMS)], a_v.at[pl.ds(0, REM_ATOMS)]
        )
        _tile_body(a_v, u_v, o_v, REM_NGROUPS)
        pltpu.sync_copy(
            o_v.at[pl.ds(0, REM_ATOMS * N_COLS)],
            out_hbm.at[pl.ds(REM_BASE * N_COLS, REM_ATOMS * N_COLS)],
        )


def _compiler_params():
    cp = pltpu.CompilerParams()
    if "needs_layout_passes" in pltpu.CompilerParams.__dataclass_fields__:
        cp = dataclasses.replace(cp, needs_layout_passes=False)
    return cp


@jax.jit
def _embed(atomic_numbers, u):
    mesh = plsc.VectorSubcoreMesh(core_axis_name="c", subcore_axis_name="s")
    run = pl.kernel(
        _sc_kernel,
        out_type=jax.ShapeDtypeStruct((N_ATOMS * N_COLS,), jnp.float32),
        mesh=mesh,
        compiler_params=_compiler_params(),
        scratch_types=[
            pltpu.VMEM((128,), jnp.float32),
            pltpu.VMEM((CHUNK,), jnp.int32),
            pltpu.VMEM((CHUNK * N_COLS,), jnp.float32),
        ],
    )
    flat = run(atomic_numbers, u)
    return flat.reshape(N_ATOMS, N_COLS)


def kernel(atomic_numbers):
    return _embed(atomic_numbers, jnp.asarray(_U))


# speedup vs baseline: 16.5903x; 4.2328x over previous
import dataclasses
import functools

import numpy as np
import jax
import jax.numpy as jnp
from jax import lax
from jax.experimental import pallas as pl
from jax.experimental.pallas import tpu as pltpu
from jax.experimental.pallas import tpu_sc as plsc

N_ATOMS = 100000
N_COLS = 6
LANES = 16
NUM_WORKERS = 32

N_TILES = 782
TPW = 25
LAST_TPW = N_TILES - 31 * TPW
Z_VALS = (6, 7, 8, 15, 16)


def _worker_body(a_ref, o_ref, ntiles):
    one = jnp.full((LANES,), 1.0, jnp.float32)
    zero = jnp.zeros((LANES,), jnp.float32)

    @pl.loop(0, ntiles)
    def _(t):
        for l in range(8):
            a16 = a_ref[pl.ds(t * 128 + l * LANES, LANES)]
            vals = [jnp.where(a16 == z, one, zero) for z in Z_VALS]
            v5 = one - (vals[0] + vals[1] + vals[2] + vals[3] + vals[4])
            vals.append(v5)
            for j in range(N_COLS):
                o_ref[t, j, pl.ds(l * LANES, LANES)] = vals[j]
            o_ref[t, 6, pl.ds(l * LANES, LANES)] = zero
            o_ref[t, 7, pl.ds(l * LANES, LANES)] = zero


def _sc_kernel(a_hbm, out_hbm, a_v, o_v):
    wid = lax.axis_index("s") * 2 + lax.axis_index("c")

    @pl.when(wid < 31)
    def _():
        base = wid * (TPW * 128)
        pltpu.sync_copy(a_hbm.at[pl.ds(base, TPW * 128)], a_v)
        _worker_body(a_v, o_v, TPW)
        pltpu.sync_copy(o_v, out_hbm.at[pl.ds(wid * TPW, TPW)])

    @pl.when(wid == 31)
    def _():
        base = 31 * (TPW * 128)
        pltpu.sync_copy(
            a_hbm.at[pl.ds(base, N_ATOMS - base)],
            a_v.at[pl.ds(0, N_ATOMS - base)],
        )
        _worker_body(a_v, o_v, LAST_TPW)
        pltpu.sync_copy(
            o_v.at[pl.ds(0, LAST_TPW)], out_hbm.at[pl.ds(31 * TPW, LAST_TPW)]
        )


def _compiler_params():
    cp = pltpu.CompilerParams()
    if "needs_layout_passes" in pltpu.CompilerParams.__dataclass_fields__:
        cp = dataclasses.replace(cp, needs_layout_passes=False)
    return cp


@jax.jit
def _embed(atomic_numbers):
    mesh = plsc.VectorSubcoreMesh(core_axis_name="c", subcore_axis_name="s")
    run = pl.kernel(
        _sc_kernel,
        out_type=jax.ShapeDtypeStruct((N_TILES, 8, 128), jnp.float32),
        mesh=mesh,
        compiler_params=_compiler_params(),
        scratch_types=[
            pltpu.VMEM((TPW * 128,), jnp.int32),
            pltpu.VMEM((TPW, 8, 128), jnp.float32),
        ],
    )
    t = run(atomic_numbers)
    return t.transpose(1, 0, 2).reshape(8, N_TILES * 128)[:N_COLS, :N_ATOMS].T


def kernel(atomic_numbers):
    return _embed(atomic_numbers)
